# baseline (speedup 1.0000x reference)
import functools

import jax
import jax.numpy as jnp
from jax import lax
from jax.experimental import pallas as pl
from jax.experimental.pallas import tpu as pltpu
from jax.experimental.pallas import tpu_sc as plsc

_NUM_CLASSES = 1000
_DIM = 999
_DIM_PAD = 1024
_BATCH = 16384

_NC = 2
_NS = 16
_NW = _NC * _NS
_B_PER_W = _BATCH // _NW
_CHUNK = 16
_NCHUNK = _B_PER_W // _CHUNK
_NVEC = 62
_TAIL = _DIM - 16


def _gather_body(y_hbm, cb_hbm, out_hbm, idx_v, bufp, bufd, gsem, wsem):
    n_chunk = y_hbm.shape[0] // _NW
    b_per_w = n_chunk * _CHUNK
    wid = lax.axis_index("s") * _NC + lax.axis_index("c")
    base = wid * b_per_w
    pltpu.sync_copy(y_hbm.at[pl.ds(wid * n_chunk, n_chunk)], idx_v)

    def start_gather(c, s):
        pltpu.async_copy(cb_hbm.at[idx_v.at[c]], bufp.at[s], gsem.at[s])

    def compact(s):
        @plsc.parallel_loop(0, _CHUNK, unroll=1)
        def compact_row(r):
            for k in range(_NVEC):
                bufd[s, r, pl.ds(16 * k, 16)] = bufp[s, r, pl.ds(16 * k, 16)]
            bufd[s, r, pl.ds(_TAIL, 16)] = bufp[s, r, pl.ds(_TAIL, 16)]

    start_gather(0, 0)
    start_gather(1, 1)

    def chunk_pair(i, carry):
        for b in range(2):
            c = 2 * i + b
            pltpu.make_async_copy(
                cb_hbm.at[idx_v.at[c]], bufp.at[b], gsem.at[b]
            ).wait()

            @pl.when(i > 0)
            def _wait_prev_writeback():
                pltpu.make_async_copy(
                    bufd.at[b], out_hbm.at[pl.ds(base, _CHUNK)], wsem.at[b]
                ).wait()

            compact(b)
            pltpu.async_copy(
                bufd.at[b],
                out_hbm.at[pl.ds(base + c * _CHUNK, _CHUNK)],
                wsem.at[b],
            )

            @pl.when(i < n_chunk // 2 - 1)
            def _start_next_gather():
                start_gather(c + 2, b)

        return carry

    lax.fori_loop(0, n_chunk // 2, chunk_pair, 0)
    for b in range(2):
        pltpu.make_async_copy(
            bufd.at[b], out_hbm.at[pl.ds(base, _CHUNK)], wsem.at[b]
        ).wait()


@jax.jit
def _lookup(y, codebook):
    n_chunk = y.shape[0] // _NW
    batch = y.shape[0] * _CHUNK
    mesh = plsc.VectorSubcoreMesh(core_axis_name="c", subcore_axis_name="s")
    return pl.kernel(
        _gather_body,
        out_type=jax.ShapeDtypeStruct((batch, _DIM), jnp.float32),
        mesh=mesh,
        scratch_types=[
            pltpu.VMEM((n_chunk, _CHUNK), jnp.int32),
            pltpu.VMEM((2, _CHUNK, _DIM_PAD), jnp.float32),
            pltpu.VMEM((2, _CHUNK, _DIM), jnp.float32),
            pltpu.SemaphoreType.DMA((2,)),
            pltpu.SemaphoreType.DMA((2,)),
        ],
        compiler_params=pltpu.CompilerParams(use_tc_tiling_on_sc=True),
    )(y, codebook)


def kernel(y, codebook):
    cb = jnp.pad(codebook, ((0, 0), (0, _DIM_PAD - _DIM)))
    yi = y.astype(jnp.int32)
    halves = []
    for h in range(2):
        yh = yi[h * (_BATCH // 2):(h + 1) * (_BATCH // 2)]
        y2 = yh.reshape(-1, _CHUNK)
        halves.append(_lookup(y2, cb))
    return jnp.concatenate(halves, axis=0)


# speedup vs baseline: 1.2827x; 1.2827x over previous
import functools

import jax
import jax.numpy as jnp
from jax import lax
from jax.experimental import pallas as pl
from jax.experimental.pallas import tpu as pltpu
from jax.experimental.pallas import tpu_sc as plsc

_NUM_CLASSES = 1000
_DIM = 999
_DIM_PAD = 1024
_BATCH = 16384

_NC = 2
_NS = 16
_NW = _NC * _NS
_B_PER_W = _BATCH // _NW
_CHUNK = 16
_NCHUNK = _B_PER_W // _CHUNK
_NVEC = 62
_TAIL = _DIM - 16


def _gather_body(y_hbm, cb_hbm, out_hbm, idx_v, bufp, bufd, gsem, wsem):
    wid = lax.axis_index("s") * _NC + lax.axis_index("c")
    base = wid * _B_PER_W
    pltpu.sync_copy(y_hbm.at[pl.ds(wid * _NCHUNK, _NCHUNK)], idx_v)

    def start_gather(c, s):
        pltpu.async_copy(cb_hbm.at[idx_v.at[c]], bufp.at[s], gsem.at[s])

    def compact(s):
        @plsc.parallel_loop(0, _CHUNK, unroll=2)
        def compact_row(r):
            for k in range(_NVEC):
                bufd[s, r, pl.ds(16 * k, 16)] = bufp[s, r, pl.ds(16 * k, 16)]
            bufd[s, r, pl.ds(_TAIL, 16)] = bufp[s, r, pl.ds(_TAIL, 16)]

    start_gather(0, 0)
    start_gather(1, 1)

    def chunk_pair(i, carry):
        for b in range(2):
            c = 2 * i + b
            pltpu.make_async_copy(
                cb_hbm.at[idx_v.at[c]], bufp.at[b], gsem.at[b]
            ).wait()

            @pl.when(i > 0)
            def _wait_prev_writeback():
                pltpu.make_async_copy(
                    bufd.at[b], out_hbm.at[pl.ds(base, _CHUNK)], wsem.at[b]
                ).wait()

            compact(b)
            pltpu.async_copy(
                bufd.at[b],
                out_hbm.at[pl.ds(base + c * _CHUNK, _CHUNK)],
                wsem.at[b],
            )

            @pl.when(i < _NCHUNK // 2 - 1)
            def _start_next_gather():
                start_gather(c + 2, b)

        return carry

    lax.fori_loop(0, _NCHUNK // 2, chunk_pair, 0)
    for b in range(2):
        pltpu.make_async_copy(
            bufd.at[b], out_hbm.at[pl.ds(base, _CHUNK)], wsem.at[b]
        ).wait()


@jax.jit
def _lookup(y, codebook):
    mesh = plsc.VectorSubcoreMesh(core_axis_name="c", subcore_axis_name="s")
    return pl.kernel(
        _gather_body,
        out_type=jax.ShapeDtypeStruct((_BATCH, _DIM), jnp.float32),
        mesh=mesh,
        scratch_types=[
            pltpu.VMEM((_NCHUNK, _CHUNK), jnp.int32),
            pltpu.VMEM((2, _CHUNK, _DIM_PAD), jnp.float32),
            pltpu.VMEM((2, _CHUNK, _DIM), jnp.float32),
            pltpu.SemaphoreType.DMA((2,)),
            pltpu.SemaphoreType.DMA((2,)),
        ],
        compiler_params=pltpu.CompilerParams(use_tc_tiling_on_sc=True),
    )(y, codebook)


def kernel(y, codebook):
    y2 = y.astype(jnp.int32).reshape(_NW * _NCHUNK, _CHUNK)
    cb = jnp.pad(codebook, ((0, 0), (0, _DIM_PAD - _DIM)))
    return _lookup(y2, cb)


# speedup vs baseline: 1.2910x; 1.0065x over previous
import functools

import jax
import jax.numpy as jnp
from jax import lax
from jax.experimental import pallas as pl
from jax.experimental.pallas import tpu as pltpu
from jax.experimental.pallas import tpu_sc as plsc

_NUM_CLASSES = 1000
_DIM = 999
_DIM_PAD = 1024
_BATCH = 16384

_NC = 2
_NS = 16
_NW = _NC * _NS
_B_PER_W = _BATCH // _NW
_CHUNK = 16
_NCHUNK = _B_PER_W // _CHUNK
_NVEC = 62
_TAIL = _DIM - 16


def _gather_body(y_hbm, cb_hbm, out_hbm, idx_v, bufp, bufd, gsem, wsem):
    wid = lax.axis_index("s") * _NC + lax.axis_index("c")
    base = wid * _B_PER_W
    pltpu.sync_copy(y_hbm.at[pl.ds(wid * _NCHUNK, _NCHUNK)], idx_v)

    def start_gather(c, s):
        pltpu.async_copy(cb_hbm.at[idx_v.at[c]], bufp.at[s], gsem.at[s])

    def compact(s, d):
        @plsc.parallel_loop(0, _CHUNK, unroll=1)
        def compact_row(r):
            for k in range(_NVEC):
                bufd[d, r, pl.ds(16 * k, 16)] = bufp[s, r, pl.ds(16 * k, 16)]
            bufd[d, r, pl.ds(_TAIL, 16)] = bufp[s, r, pl.ds(_TAIL, 16)]

    for b in range(4):
        start_gather(b, b)

    def chunk_quad(i, carry):
        for b in range(4):
            c = 4 * i + b
            d = b % 2
            pltpu.make_async_copy(
                cb_hbm.at[idx_v.at[c]], bufp.at[b], gsem.at[b]
            ).wait()

            @pl.when((i > 0) | (b >= 2))
            def _wait_prev_writeback():
                pltpu.make_async_copy(
                    bufd.at[d], out_hbm.at[pl.ds(base, _CHUNK)], wsem.at[d]
                ).wait()

            compact(b, d)
            pltpu.async_copy(
                bufd.at[d],
                out_hbm.at[pl.ds(base + c * _CHUNK, _CHUNK)],
                wsem.at[d],
            )

            @pl.when(i < _NCHUNK // 4 - 1)
            def _start_next_gather():
                start_gather(c + 4, b)

        return carry

    lax.fori_loop(0, _NCHUNK // 4, chunk_quad, 0)
    for b in range(2):
        pltpu.make_async_copy(
            bufd.at[b], out_hbm.at[pl.ds(base, _CHUNK)], wsem.at[b]
        ).wait()


@jax.jit
def _lookup(y, codebook):
    mesh = plsc.VectorSubcoreMesh(core_axis_name="c", subcore_axis_name="s")
    return pl.kernel(
        _gather_body,
        out_type=jax.ShapeDtypeStruct((_BATCH, _DIM), jnp.float32),
        mesh=mesh,
        scratch_types=[
            pltpu.VMEM((_NCHUNK, _CHUNK), jnp.int32),
            pltpu.VMEM((4, _CHUNK, _DIM_PAD), jnp.float32),
            pltpu.VMEM((2, _CHUNK, _DIM), jnp.float32),
            pltpu.SemaphoreType.DMA((4,)),
            pltpu.SemaphoreType.DMA((2,)),
        ],
        compiler_params=pltpu.CompilerParams(use_tc_tiling_on_sc=True),
    )(y, codebook)


def kernel(y, codebook):
    y2 = y.astype(jnp.int32).reshape(_NW * _NCHUNK, _CHUNK)
    cb = jnp.pad(codebook, ((0, 0), (0, _DIM_PAD - _DIM)))
    return _lookup(y2, cb)


# speedup vs baseline: 1.2956x; 1.0036x over previous
import functools

import jax
import jax.numpy as jnp
from jax import lax
from jax.experimental import pallas as pl
from jax.experimental.pallas import tpu as pltpu
from jax.experimental.pallas import tpu_sc as plsc

_NUM_CLASSES = 1000
_DIM = 999
_DIM_PAD = 1024
_BATCH = 16384

_NC = 2
_NS = 16
_NW = _NC * _NS
_B_PER_W = _BATCH // _NW
_CHUNK = 16
_NCHUNK = _B_PER_W // _CHUNK
_NVEC = 62
_TAIL = _DIM - 16


def _gather_body(y_hbm, cb_hbm, out_hbm, idx_v, bufp, bufd, gsem, wsem):
    wid = lax.axis_index("s") * _NC + lax.axis_index("c")
    base = wid * _B_PER_W
    pltpu.sync_copy(y_hbm.at[pl.ds(wid * _NCHUNK, _NCHUNK)], idx_v)

    def start_gather(c, s):
        pltpu.async_copy(cb_hbm.at[idx_v.at[c]], bufp.at[s], gsem.at[s])

    def compact(s):
        @plsc.parallel_loop(0, _CHUNK, unroll=1)
        def compact_row(r):
            for k in range(_NVEC):
                bufd[s, r, pl.ds(16 * k, 16)] = bufp[s, r, pl.ds(16 * k, 16)]
            bufd[s, r, pl.ds(_TAIL, 16)] = bufp[s, r, pl.ds(_TAIL, 16)]

    start_gather(0, 0)
    start_gather(1, 1)

    def chunk_pair(i, carry):
        for b in range(2):
            c = 2 * i + b
            pltpu.make_async_copy(
                cb_hbm.at[idx_v.at[c]], bufp.at[b], gsem.at[b]
            ).wait()

            @pl.when(i > 0)
            def _wait_prev_writeback():
                pltpu.make_async_copy(
                    bufd.at[b], out_hbm.at[pl.ds(base, _CHUNK)], wsem.at[b]
                ).wait()

            compact(b)
            pltpu.async_copy(
                bufd.at[b],
                out_hbm.at[pl.ds(base + c * _CHUNK, _CHUNK)],
                wsem.at[b],
            )

            @pl.when(i < _NCHUNK // 2 - 1)
            def _start_next_gather():
                start_gather(c + 2, b)

        return carry

    lax.fori_loop(0, _NCHUNK // 2, chunk_pair, 0)
    for b in range(2):
        pltpu.make_async_copy(
            bufd.at[b], out_hbm.at[pl.ds(base, _CHUNK)], wsem.at[b]
        ).wait()


@jax.jit
def _lookup(y, codebook):
    mesh = plsc.VectorSubcoreMesh(core_axis_name="c", subcore_axis_name="s")
    return pl.kernel(
        _gather_body,
        out_type=jax.ShapeDtypeStruct((_BATCH, _DIM), jnp.float32),
        mesh=mesh,
        scratch_types=[
            pltpu.VMEM((_NCHUNK, _CHUNK), jnp.int32),
            pltpu.VMEM((2, _CHUNK, _DIM_PAD), jnp.float32),
            pltpu.VMEM((2, _CHUNK, _DIM), jnp.float32),
            pltpu.SemaphoreType.DMA((2,)),
            pltpu.SemaphoreType.DMA((2,)),
        ],
        compiler_params=pltpu.CompilerParams(use_tc_tiling_on_sc=True),
    )(y, codebook)


def kernel(y, codebook):
    y2 = y.astype(jnp.int32).reshape(_NW * _NCHUNK, _CHUNK)
    cb = jnp.pad(codebook, ((0, 0), (0, _DIM_PAD - _DIM)))
    return _lookup(y2, cb)
